# Initial kernel scaffold; baseline (speedup 1.0000x reference)
import jax
import jax.numpy as jnp
from jax.experimental import pallas as pl


def kernel(x, edge_index, agg_scatter_index_0, agg_node_index_0, agg_scatter_index_1, agg_node_index_1, eps, beta1, beta2, beta3, W_head, b_head):
    raise NotImplementedError("write your pallas kernel here")


# speedup vs baseline: 16.2912x; 16.2912x over previous
import jax
import jax.numpy as jnp
from jax import lax
from jax.experimental import pallas as pl
from jax.experimental.pallas import tpu as pltpu
from jax.experimental.pallas import tpu_sc as plsc

N = 10000
D = 128
E = 320000
W = 80
WPS = 2
SLOT = W * WPS
NG = E // SLOT
NC = 2
NS = 16
NWK = NC * NS
TRIPS = (NG + NWK - 1) // NWK
RPT = 624
TAIL = N - NS * RPT


UNROLL = 4
OUTER = (TRIPS + UNROLL - 1) // UNROLL


def _sc_hop_body(h_hbm, si_hbm, ni_hbm, z2_hbm, s_out, c_out0, c_out1,
                 si_v0, si_v1, si_v2, si_v3, ni_v0, ni_v1, ni_v2, ni_v3,
                 rows_v0, rows_v1, ones_v, cbuf_v, S_sh, cnt_sh,
                 isem0, isem1, isem2, isem3, gsem0, gsem1, gsem2, gsem3,
                 ssem0, ssem1):
    c = lax.axis_index("c")
    s = lax.axis_index("s")
    wid = s * NC + c
    si_v = [si_v0, si_v1, si_v2, si_v3]
    ni_v = [ni_v0, ni_v1, ni_v2, ni_v3]
    rows_v = [rows_v0, rows_v1]
    isem = [isem0, isem1, isem2, isem3]
    gsem = [[gsem0, gsem1], [gsem2, gsem3]]
    ssem = [ssem0, ssem1]

    for i in range(W // 16):
        ones_v[pl.ds(i * 16, 16)] = jnp.full((16,), 1.0, jnp.float32)
    for i in range(RPT // 16):
        cbuf_v[pl.ds(i * 16, 16)] = jnp.zeros((16,), jnp.float32)

    pltpu.sync_copy(z2_hbm.at[pl.ds(s * RPT, RPT)], S_sh.at[pl.ds(s * RPT, RPT)])
    pltpu.sync_copy(cbuf_v, cnt_sh.at[pl.ds(s * RPT, RPT)])

    @pl.when(s == 0)
    def _():
        pltpu.sync_copy(z2_hbm.at[pl.ds(NS * RPT, TAIL)],
                        S_sh.at[pl.ds(NS * RPT, TAIL)])
        pltpu.sync_copy(cbuf_v.at[pl.ds(0, TAIL)],
                        cnt_sh.at[pl.ds(NS * RPT, TAIL)])

    def issue_idx(g, slot):
        for r in range(WPS):
            off = g * SLOT + r * W
            pltpu.async_copy(si_hbm.at[pl.ds(off, W)], si_v[slot].at[r],
                             isem[slot])
            pltpu.async_copy(ni_hbm.at[pl.ds(off, W)], ni_v[slot].at[r],
                             isem[slot])

    def wait_idx(g, slot):
        for r in range(WPS):
            off = g * SLOT + r * W
            pltpu.make_async_copy(si_hbm.at[pl.ds(off, W)], si_v[slot].at[r],
                                  isem[slot]).wait()
            pltpu.make_async_copy(ni_hbm.at[pl.ds(off, W)], ni_v[slot].at[r],
                                  isem[slot]).wait()

    def wait_scatter(b2):
        for r in range(WPS):
            pltpu.make_async_copy(rows_v[b2].at[pl.ds(r * W, W)],
                                  S_sh.at[ni_v[0].at[r]], ssem[b2]).wait()
            pltpu.make_async_copy(ones_v, cnt_sh.at[ni_v[0].at[r]],
                                  ssem[b2]).wait()

    issue_idx(wid, 0)
    issue_idx(wid + NWK, 1)

    plsc.subcore_barrier()

    def outer(i, carry):
        for j in range(UNROLL):
            b2 = j % 2
            k = i * UNROLL + j
            g = wid + k * NWK

            @pl.when(g < NG)
            def _(j=j, b2=b2, k=k, g=g):
                wait_idx(g, j)

                @pl.when(k >= 2)
                def _():
                    wait_scatter(b2)

                for r in range(WPS):
                    pltpu.async_copy(h_hbm.at[si_v[j].at[r]],
                                     rows_v[b2].at[pl.ds(r * W, W)],
                                     gsem[b2][r])

                g2 = g + 2 * NWK

                @pl.when(g2 < NG)
                def _():
                    issue_idx(g2, (j + 2) % UNROLL)

                for r in range(WPS):
                    pltpu.make_async_copy(h_hbm.at[si_v[j].at[r]],
                                          rows_v[b2].at[pl.ds(r * W, W)],
                                          gsem[b2][r]).wait()
                    pltpu.async_copy(rows_v[b2].at[pl.ds(r * W, W)],
                                     S_sh.at[ni_v[j].at[r]], ssem[b2],
                                     add=True)
                    pltpu.async_copy(ones_v, cnt_sh.at[ni_v[j].at[r]],
                                     ssem[b2], add=True)

        return carry

    lax.fori_loop(0, OUTER, outer, 0)

    wait_scatter(0)
    wait_scatter(1)

    plsc.subcore_barrier()

    pltpu.sync_copy(S_sh.at[pl.ds(s * RPT, RPT)], s_out.at[c, pl.ds(s * RPT, RPT)])
    pltpu.sync_copy(cnt_sh.at[pl.ds(s * RPT, RPT)], cbuf_v)

    @pl.when(c == 0)
    def _():
        pltpu.sync_copy(cbuf_v, c_out0.at[pl.ds(s * RPT, RPT)])

    @pl.when(c == 1)
    def _():
        pltpu.sync_copy(cbuf_v, c_out1.at[pl.ds(s * RPT, RPT)])

    @pl.when(s == 0)
    def _():
        pltpu.sync_copy(S_sh.at[pl.ds(NS * RPT, TAIL)],
                        s_out.at[c, pl.ds(NS * RPT, TAIL)])

        pltpu.sync_copy(cnt_sh.at[pl.ds(NS * RPT, TAIL)],
                        cbuf_v.at[pl.ds(0, TAIL)])

        @pl.when(c == 0)
        def _():
            pltpu.sync_copy(cbuf_v.at[pl.ds(0, TAIL)],
                            c_out0.at[pl.ds(NS * RPT, TAIL)])

        @pl.when(c == 1)
        def _():
            pltpu.sync_copy(cbuf_v.at[pl.ds(0, TAIL)],
                            c_out1.at[pl.ds(NS * RPT, TAIL)])


_sc_hop = pl.kernel(
    _sc_hop_body,
    out_type=[
        jax.ShapeDtypeStruct((NC, N, D), jnp.float32),
        jax.ShapeDtypeStruct((N,), jnp.float32),
        jax.ShapeDtypeStruct((N,), jnp.float32),
    ],
    mesh=plsc.VectorSubcoreMesh(core_axis_name="c", subcore_axis_name="s",
                                num_cores=NC, num_subcores=NS),
    scratch_types=(
        [pltpu.VMEM((WPS, W), jnp.int32)] * 8
        + [pltpu.VMEM((SLOT, D), jnp.float32)] * 2
        + [
            pltpu.VMEM((W,), jnp.float32),
            pltpu.VMEM((RPT,), jnp.float32),
            pltpu.VMEM_SHARED((N, D), jnp.float32),
            pltpu.VMEM_SHARED((N,), jnp.float32),
        ]
        + [pltpu.SemaphoreType.DMA] * 10
    ),
)


R = 2000


def _hop_update(sp_ref, c0_ref, c1_ref, x_ref, par_ref, off):
    a1 = par_ref[off + 0]
    b2 = par_ref[off + 1]
    a3 = par_ref[off + 2]
    S = sp_ref[0] + sp_ref[1]
    cnt = c0_ref[...] + c1_ref[...]
    xb = x_ref[...]
    summed = a1 * S + cnt * xb
    h = jnp.where(cnt > 0.0, summed, xb)
    return a3 * (h + 1.0 + b2 * xb)


def _combine_body(sp_ref, c0_ref, c1_ref, x_ref, par_ref, o_ref):
    o_ref[...] = _hop_update(sp_ref, c0_ref, c1_ref, x_ref, par_ref, 0)


def _final_body(sp_ref, c0_ref, c1_ref, x_ref, h1_ref, par_ref, w_ref, b_ref,
                o_ref):
    h2 = _hop_update(sp_ref, c0_ref, c1_ref, x_ref, par_ref, 0)
    ae = par_ref[3]
    acc = ae * x_ref[...] + h1_ref[...] + h2
    o_ref[...] = jnp.dot(acc, w_ref[...],
                         preferred_element_type=jnp.float32) + b_ref[...]


_combine = pl.pallas_call(
    _combine_body,
    grid=(N // R,),
    in_specs=[
        pl.BlockSpec((NC, R, D), lambda i: (0, i, 0)),
        pl.BlockSpec((R, 1), lambda i: (i, 0)),
        pl.BlockSpec((R, 1), lambda i: (i, 0)),
        pl.BlockSpec((R, D), lambda i: (i, 0)),
        pl.BlockSpec(memory_space=pltpu.SMEM),
    ],
    out_specs=pl.BlockSpec((R, D), lambda i: (i, 0)),
    out_shape=jax.ShapeDtypeStruct((N, D), jnp.float32),
)

_final = pl.pallas_call(
    _final_body,
    grid=(N // R,),
    in_specs=[
        pl.BlockSpec((NC, R, D), lambda i: (0, i, 0)),
        pl.BlockSpec((R, 1), lambda i: (i, 0)),
        pl.BlockSpec((R, 1), lambda i: (i, 0)),
        pl.BlockSpec((R, D), lambda i: (i, 0)),
        pl.BlockSpec((R, D), lambda i: (i, 0)),
        pl.BlockSpec(memory_space=pltpu.SMEM),
        pl.BlockSpec((D, D), lambda i: (0, 0)),
        pl.BlockSpec((1, D), lambda i: (0, 0)),
    ],
    out_specs=pl.BlockSpec((R, D), lambda i: (i, 0)),
    out_shape=jax.ShapeDtypeStruct((N, D), jnp.float32),
)


def kernel(x, edge_index, agg_scatter_index_0, agg_node_index_0,
           agg_scatter_index_1, agg_node_index_1,
           eps, beta1, beta2, beta3, W_head, b_head):
    del edge_index
    z2 = jnp.zeros((N, D), jnp.float32)

    sp0, cp0a, cp0b = _sc_hop(x, agg_scatter_index_0, agg_node_index_0, z2)
    par0 = jnp.stack([1.0 + beta1[0], beta2[0], 1.0 + beta3[0],
                      jnp.float32(0.0)])
    h1 = _combine(sp0, cp0a[:, None], cp0b[:, None], x, par0)

    sp1, cp1a, cp1b = _sc_hop(h1, agg_scatter_index_1, agg_node_index_1, z2)
    par1 = jnp.stack([1.0 + beta1[1], beta2[1], 1.0 + beta3[1], 1.0 + eps[0]])
    return _final(sp1, cp1a[:, None], cp1b[:, None], x, h1, par1,
                  W_head, b_head[None, :])
